# baseline (speedup 1.0000x reference)
import functools
import math

import jax
import jax.numpy as jnp
import numpy as np
from jax import lax
from jax.experimental import pallas as pl
from jax.experimental.pallas import tpu as pltpu
from jax.experimental.pallas import tpu_sc as plsc

VOCAB = 100000
D_MODEL = 768
NUM_LANG = 4
MAX_LEN = 4096
B = 4
S = 4096
SCALE = math.sqrt(D_MODEL)

_NW = 32
_SPW = S // _NW
_K = 32
_NSC = _SPW // _K
_L = 16
_NJ = D_MODEL // _L
_JB = 8


def _pe_np():
    pos = np.arange(MAX_LEN, dtype=np.float32)[:, None]
    div = np.exp(
        np.arange(0, D_MODEL, 2, dtype=np.float32)
        * np.float32(-math.log(10000.0) / D_MODEL)
    ).astype(np.float32)
    pe = np.zeros((MAX_LEN, D_MODEL), dtype=np.float32)
    pe[:, 0::2] = np.sin(pos * div)
    pe[:, 1::2] = np.cos(pos * div)
    return pe


_PE = _pe_np()
_GATHER_DN = lax.GatherDimensionNumbers(
    offset_dims=(), collapsed_slice_dims=(0,), start_index_map=(0,)
)


def _lane_splat(vec, lane):
    idx = jnp.full((16, 1), lane, jnp.int32)
    return lax.gather(
        vec, idx, _GATHER_DN, slice_sizes=(1,),
        mode=lax.GatherScatterMode.PROMISE_IN_BOUNDS,
    )


def _lang_tab_body(wl_ref, wp_ref, out_ref):
    out_ref[...] = lax.dot_general(
        wl_ref[...],
        wp_ref[...],
        (((1,), (1,)), ((), ())),
        preferred_element_type=jnp.float32,
    )


def _lang_tab(W_lang, W_proj):
    return pl.pallas_call(
        _lang_tab_body,
        out_shape=jax.ShapeDtypeStruct((NUM_LANG, D_MODEL), jnp.float32),
    )(W_lang, W_proj)


_mesh = plsc.VectorSubcoreMesh(core_axis_name="c", subcore_axis_name="s")


_NQ = B * _NSC


@functools.partial(
    pl.kernel,
    mesh=_mesh,
    out_type=jax.ShapeDtypeStruct((B * S, D_MODEL), jnp.float32),
    scratch_types=[
        pltpu.VMEM((B * _SPW,), jnp.int32),
        pltpu.VMEM((B * _SPW,), jnp.int32),
        pltpu.VMEM((_K, D_MODEL), jnp.float32),
        pltpu.VMEM((_K, D_MODEL), jnp.float32),
        pltpu.VMEM((2 * _K, D_MODEL), jnp.float32),
        pltpu.VMEM((NUM_LANG, D_MODEL), jnp.float32),
        pltpu.VMEM((_K, _L), jnp.int32),
        pltpu.SemaphoreType.DMA,
        pltpu.SemaphoreType.DMA,
        pltpu.SemaphoreType.DMA,
        pltpu.SemaphoreType.DMA,
        pltpu.SemaphoreType.DMA,
    ],
)
def _sc_embed(tok_ids, lang_ids, w_tok, lang_tab, pe, out,
              tokidx, langidx, buf_a, buf_b, pebuf, lang_v, lidsplat,
              sem_ga, sem_gb, sem_oa, sem_ob, sem_pe):
    cid = lax.axis_index("c")
    sid = lax.axis_index("s")
    wid = sid * 2 + cid
    wbase = wid * _SPW

    def q_coords(q):
        sc = q // _NSC
        b = q - sc * _NSC
        idx_off = b * _SPW + sc * _K
        t0 = b * S + wbase + sc * _K
        return sc, b, idx_off, t0

    def gather_desc(q, buf, sem):
        _, _, idx_off, _ = q_coords(q)
        return pltpu.make_async_copy(
            w_tok.at[tokidx.at[pl.ds(idx_off, _K)]], buf, sem
        )

    def out_desc(q, buf, sem):
        _, _, _, t0 = q_coords(q)
        return pltpu.make_async_copy(buf, out.at[pl.ds(t0, _K)], sem)

    def pe_desc(sc, sem):
        src = pe.at[pl.ds(wbase + sc * _K, _K)]
        dst = pebuf.at[pl.ds(lax.rem(sc, 2) * _K, _K)]
        return pltpu.make_async_copy(src, dst, sem)

    pltpu.sync_copy(lang_tab, lang_v)

    def idcopy(b, _):
        pltpu.sync_copy(tok_ids.at[pl.ds(b * S + wbase, _SPW)],
                        tokidx.at[pl.ds(b * _SPW, _SPW)])
        pltpu.sync_copy(lang_ids.at[pl.ds(b * S + wbase, _SPW)],
                        langidx.at[pl.ds(b * _SPW, _SPW)])
        return _

    lax.fori_loop(0, B, idcopy, None)
    pe_desc(0, sem_pe).start()
    gather_desc(0, buf_a, sem_ga).start()

    def step(q, buf_x, sem_gx, sem_ox, buf_y, sem_gy, sem_oy):
        sc, b, idx_off, t0 = q_coords(q)

        @pl.when(jnp.logical_and(q >= 1, q + 1 < _NQ))
        def _wait_oy():
            out_desc(q - 1, buf_y, sem_oy).wait()

        @pl.when(q + 1 < _NQ)
        def _issue_next():
            gather_desc(q + 1, buf_y, sem_gy).start()

        gather_desc(q, buf_x, sem_gx).wait()

        @pl.when(b == 0)
        def _pe_edge():
            pe_desc(sc, sem_pe).wait()

            @pl.when(sc + 1 < _NSC)
            def _pe_next():
                pe_desc(sc + 1, sem_pe).start()

        def splat_grp(g, _):
            lvec = langidx[pl.ds(idx_off + g * _L, _L)]
            for i16 in range(_L):
                lidsplat.at[g * _L + i16][:] = _lane_splat(lvec, i16)
            return _

        lax.fori_loop(0, _K // _L, splat_grp, None)

        pe_row0 = lax.rem(sc, 2) * _K
        for jb in range(_NJ // _JB):
            rows = [
                [lang_v.at[l][pl.ds((jb * _JB + j) * _L, _L)] for l in range(NUM_LANG)]
                for j in range(_JB)
            ]

            def tok_loop(i, _, jb=jb, rows=rows):
                lid = lidsplat.at[i][:]
                m0 = lid == 0
                m1 = lid == 1
                m2 = lid == 2
                for j in range(_JB):
                    jj = jb * _JB + j
                    t = buf_x.at[i][pl.ds(jj * _L, _L)]
                    p = pebuf.at[pe_row0 + i][pl.ds(jj * _L, _L)]
                    r = jnp.where(
                        m0, rows[j][0],
                        jnp.where(m1, rows[j][1],
                                  jnp.where(m2, rows[j][2], rows[j][3])),
                    )
                    buf_x.at[i][pl.ds(jj * _L, _L)] = t * SCALE + p + r
                return _

            lax.fori_loop(0, _K, tok_loop, None)

        out_desc(q, buf_x, sem_ox).start()

    def pair_loop(p, _):
        qa = 2 * p
        step(qa, buf_a, sem_ga, sem_oa, buf_b, sem_gb, sem_ob)
        step(qa + 1, buf_b, sem_gb, sem_ob, buf_a, sem_ga, sem_oa)
        return _

    lax.fori_loop(0, _NQ // 2, pair_loop, None)

    out_desc(_NQ - 2, buf_a, sem_oa).wait()
    out_desc(_NQ - 1, buf_b, sem_ob).wait()


def kernel(token_ids, lang_ids, W_tok, W_lang, W_proj):
    lang_tab = _lang_tab(W_lang, W_proj)
    tok_flat = token_ids.reshape(-1).astype(jnp.int32)
    lang_flat = lang_ids.reshape(-1).astype(jnp.int32)
    pe = jnp.asarray(_PE[:S])
    out = _sc_embed(tok_flat, lang_flat, W_tok, lang_tab, pe)
    return out.reshape(B, S, D_MODEL)


# speedup vs baseline: 1.8097x; 1.8097x over previous
import functools
import math

import jax
import jax.numpy as jnp
import numpy as np
from jax import lax
from jax.experimental import pallas as pl
from jax.experimental.pallas import tpu as pltpu
from jax.experimental.pallas import tpu_sc as plsc

VOCAB = 100000
D_MODEL = 768
NUM_LANG = 4
MAX_LEN = 4096
B = 4
S = 4096
SCALE = math.sqrt(D_MODEL)

_NW = 32
_SPW = S // _NW
_K = 32
_NSC = _SPW // _K
_L = 16
_NJ = D_MODEL // _L
_JB = 8


def _pe_np():
    pos = np.arange(MAX_LEN, dtype=np.float32)[:, None]
    div = np.exp(
        np.arange(0, D_MODEL, 2, dtype=np.float32)
        * np.float32(-math.log(10000.0) / D_MODEL)
    ).astype(np.float32)
    pe = np.zeros((MAX_LEN, D_MODEL), dtype=np.float32)
    pe[:, 0::2] = np.sin(pos * div)
    pe[:, 1::2] = np.cos(pos * div)
    return pe


_PE = _pe_np()
_GATHER_DN = lax.GatherDimensionNumbers(
    offset_dims=(), collapsed_slice_dims=(0,), start_index_map=(0,)
)


def _lane_splat(vec, lane):
    idx = jnp.full((16, 1), lane, jnp.int32)
    return lax.gather(
        vec, idx, _GATHER_DN, slice_sizes=(1,),
        mode=lax.GatherScatterMode.PROMISE_IN_BOUNDS,
    )


def _lang_tab_body(wl_ref, wp_ref, out_ref):
    out_ref[...] = lax.dot_general(
        wl_ref[...],
        wp_ref[...],
        (((1,), (1,)), ((), ())),
        preferred_element_type=jnp.float32,
    )


def _lang_tab(W_lang, W_proj):
    return pl.pallas_call(
        _lang_tab_body,
        out_shape=jax.ShapeDtypeStruct((NUM_LANG, D_MODEL), jnp.float32),
    )(W_lang, W_proj)


_mesh = plsc.VectorSubcoreMesh(core_axis_name="c", subcore_axis_name="s")


_NQ = B * _NSC


@functools.partial(
    pl.kernel,
    mesh=_mesh,
    out_type=jax.ShapeDtypeStruct((B * S, D_MODEL), jnp.float32),
    scratch_types=[
        pltpu.VMEM((_K,), jnp.int32),
        pltpu.VMEM((_K,), jnp.int32),
        pltpu.VMEM((_K,), jnp.int32),
        pltpu.VMEM((_K,), jnp.int32),
        pltpu.VMEM((_K, D_MODEL), jnp.float32),
        pltpu.VMEM((_K, D_MODEL), jnp.float32),
        pltpu.VMEM((_K, D_MODEL), jnp.float32),
        pltpu.VMEM((NUM_LANG, D_MODEL), jnp.float32),
        pltpu.VMEM((_K, _L), jnp.int32),
        pltpu.SemaphoreType.DMA,
        pltpu.SemaphoreType.DMA,
        pltpu.SemaphoreType.DMA,
        pltpu.SemaphoreType.DMA,
        pltpu.SemaphoreType.DMA,
    ],
)
def _sc_embed(tok_ids, lang_ids, w_tok, lang_tab, pe, out,
              toka, tokb, langa, langb, buf_a, buf_b, pebuf, lang_v, lidsplat,
              sem_ga, sem_gb, sem_oa, sem_ob, sem_pe):
    cid = lax.axis_index("c")
    sid = lax.axis_index("s")
    wid = sid * 2 + cid
    wbase = wid * _SPW

    def id_off(sc, b):
        return b * S + wbase + sc * _K

    def copy_ids(sc, b, tokx, langx):
        off = id_off(sc, b)
        pltpu.sync_copy(tok_ids.at[pl.ds(off, _K)], tokx)
        pltpu.sync_copy(lang_ids.at[pl.ds(off, _K)], langx)

    def gather_desc(tokx, buf, sem):
        return pltpu.make_async_copy(w_tok.at[tokx], buf, sem)

    def out_desc(sc, b, buf, sem):
        t0 = b * S + wbase + sc * _K
        return pltpu.make_async_copy(buf, out.at[pl.ds(t0, _K)], sem)

    def pe_desc(sc, sem):
        return pltpu.make_async_copy(pe.at[pl.ds(wbase + sc * _K, _K)], pebuf, sem)

    pltpu.sync_copy(lang_tab, lang_v)
    pe_desc(0, sem_pe).start()
    copy_ids(0, 0, toka, langa)
    gather_desc(toka, buf_a, sem_ga).start()

    def compute(buf_x, langx):
        def splat_grp(g, _):
            lvec = langx[pl.ds(g * _L, _L)]
            for i16 in range(_L):
                lidsplat.at[g * _L + i16][:] = _lane_splat(lvec, i16)
            return _

        lax.fori_loop(0, _K // _L, splat_grp, None)

        for jb in range(_NJ // _JB):
            rows = [
                [lang_v.at[l][pl.ds((jb * _JB + j) * _L, _L)] for l in range(NUM_LANG)]
                for j in range(_JB)
            ]

            def tok_loop(i, _, jb=jb, rows=rows):
                lid = lidsplat.at[i][:]
                m0 = lid == 0
                m1 = lid == 1
                m2 = lid == 2
                for j in range(_JB):
                    jj = jb * _JB + j
                    t = buf_x.at[i][pl.ds(jj * _L, _L)]
                    p = pebuf.at[i][pl.ds(jj * _L, _L)]
                    r = jnp.where(
                        m0, rows[j][0],
                        jnp.where(m1, rows[j][1],
                                  jnp.where(m2, rows[j][2], rows[j][3])),
                    )
                    buf_x.at[i][pl.ds(jj * _L, _L)] = t * SCALE + p + r
                return _

            lax.fori_loop(0, _K, tok_loop, None)

    def step(sc, b,
             buf_x, tokx, langx, sem_gx, sem_ox,
             buf_y, toky, langy, sem_gy, sem_oy):
        if b == 0:
            @pl.when(sc != 0)
            def _wait_oy():
                out_desc(sc - 1, B - 1, buf_y, sem_oy).wait()

            copy_ids(sc, 1, toky, langy)
            gather_desc(toky, buf_y, sem_gy).start()
        elif b < B - 1:
            out_desc(sc, b - 1, buf_y, sem_oy).wait()
            copy_ids(sc, b + 1, toky, langy)
            gather_desc(toky, buf_y, sem_gy).start()
        else:
            @pl.when(sc + 1 < _NSC)
            def _next_sc():
                out_desc(sc, b - 1, buf_y, sem_oy).wait()
                copy_ids(sc + 1, 0, toky, langy)
                gather_desc(toky, buf_y, sem_gy).start()

        gather_desc(tokx, buf_x, sem_gx).wait()

        if b == 0:
            pe_desc(sc, sem_pe).wait()

        compute(buf_x, langx)

        if b == B - 1:
            @pl.when(sc + 1 < _NSC)
            def _pe_next():
                pe_desc(sc + 1, sem_pe).start()

        out_desc(sc, b, buf_x, sem_ox).start()

    def sc_loop(sc, _):
        step(sc, 0, buf_a, toka, langa, sem_ga, sem_oa,
             buf_b, tokb, langb, sem_gb, sem_ob)
        step(sc, 1, buf_b, tokb, langb, sem_gb, sem_ob,
             buf_a, toka, langa, sem_ga, sem_oa)
        step(sc, 2, buf_a, toka, langa, sem_ga, sem_oa,
             buf_b, tokb, langb, sem_gb, sem_ob)
        step(sc, 3, buf_b, tokb, langb, sem_gb, sem_ob,
             buf_a, toka, langa, sem_ga, sem_oa)
        return _

    lax.fori_loop(0, _NSC, sc_loop, None)

    out_desc(_NSC - 1, 2, buf_a, sem_oa).wait()
    out_desc(_NSC - 1, 3, buf_b, sem_ob).wait()


def kernel(token_ids, lang_ids, W_tok, W_lang, W_proj):
    lang_tab = _lang_tab(W_lang, W_proj)
    tok_flat = token_ids.reshape(-1).astype(jnp.int32)
    lang_flat = lang_ids.reshape(-1).astype(jnp.int32)
    pe = jnp.asarray(_PE[:S])
    out = _sc_embed(tok_flat, lang_flat, W_tok, lang_tab, pe)
    return out.reshape(B, S, D_MODEL)


# speedup vs baseline: 1.9495x; 1.0773x over previous
import functools
import math

import jax
import jax.numpy as jnp
import numpy as np
from jax import lax
from jax.experimental import pallas as pl
from jax.experimental.pallas import tpu as pltpu
from jax.experimental.pallas import tpu_sc as plsc

VOCAB = 100000
D_MODEL = 768
NUM_LANG = 4
MAX_LEN = 4096
B = 4
S = 4096
SCALE = math.sqrt(D_MODEL)

_NW = 32
_SPW = S // _NW
_K = 32
_NSC = _SPW // _K
_L = 16
_NJ = D_MODEL // _L
_JB = 8


def _pe_np():
    pos = np.arange(MAX_LEN, dtype=np.float32)[:, None]
    div = np.exp(
        np.arange(0, D_MODEL, 2, dtype=np.float32)
        * np.float32(-math.log(10000.0) / D_MODEL)
    ).astype(np.float32)
    pe = np.zeros((MAX_LEN, D_MODEL), dtype=np.float32)
    pe[:, 0::2] = np.sin(pos * div)
    pe[:, 1::2] = np.cos(pos * div)
    return pe


_PE = _pe_np()
_GATHER_DN = lax.GatherDimensionNumbers(
    offset_dims=(), collapsed_slice_dims=(0,), start_index_map=(0,)
)


def _lane_splat(vec, lane):
    idx = jnp.full((16, 1), lane, jnp.int32)
    return lax.gather(
        vec, idx, _GATHER_DN, slice_sizes=(1,),
        mode=lax.GatherScatterMode.PROMISE_IN_BOUNDS,
    )


def _lang_tab_body(wl_ref, wp_ref, out_ref):
    out_ref[...] = lax.dot_general(
        wl_ref[...],
        wp_ref[...],
        (((1,), (1,)), ((), ())),
        preferred_element_type=jnp.float32,
    )


def _lang_tab(W_lang, W_proj):
    return pl.pallas_call(
        _lang_tab_body,
        out_shape=jax.ShapeDtypeStruct((NUM_LANG, D_MODEL), jnp.float32),
    )(W_lang, W_proj)


_mesh = plsc.VectorSubcoreMesh(core_axis_name="c", subcore_axis_name="s")


_NQ = B * _NSC


@functools.partial(
    pl.kernel,
    mesh=_mesh,
    out_type=jax.ShapeDtypeStruct((B * S, D_MODEL), jnp.float32),
    scratch_types=[
        pltpu.VMEM((_K,), jnp.int32),
        pltpu.VMEM((_K,), jnp.int32),
        pltpu.VMEM((_K,), jnp.int32),
        pltpu.VMEM((_K,), jnp.int32),
        pltpu.VMEM((_K, D_MODEL), jnp.float32),
        pltpu.VMEM((_K, D_MODEL), jnp.float32),
        pltpu.VMEM((_K, D_MODEL), jnp.float32),
        pltpu.VMEM((NUM_LANG, D_MODEL), jnp.float32),
        pltpu.VMEM((_K, _L), jnp.int32),
        pltpu.SemaphoreType.DMA,
        pltpu.SemaphoreType.DMA,
        pltpu.SemaphoreType.DMA,
        pltpu.SemaphoreType.DMA,
        pltpu.SemaphoreType.DMA,
        pltpu.SemaphoreType.DMA,
        pltpu.SemaphoreType.DMA,
    ],
)
def _sc_embed(tok_ids, lang_ids, w_tok, lang_tab, pe, out,
              toka, tokb, langa, langb, buf_a, buf_b, pebuf, lang_v, lidsplat,
              sem_ga, sem_gb, sem_oa, sem_ob, sem_pe, sem_ia, sem_ib):
    cid = lax.axis_index("c")
    sid = lax.axis_index("s")
    wid = sid * 2 + cid
    wbase = wid * _SPW

    def id_off(sc, b):
        return b * S + wbase + sc * _K

    def copy_ids(sc, b, tokx, langx):
        off = id_off(sc, b)
        pltpu.sync_copy(tok_ids.at[pl.ds(off, _K)], tokx)
        pltpu.sync_copy(lang_ids.at[pl.ds(off, _K)], langx)

    def id_descs(sc, b, tokx, langx, sem):
        off = id_off(sc, b)
        return (pltpu.make_async_copy(tok_ids.at[pl.ds(off, _K)], tokx, sem),
                pltpu.make_async_copy(lang_ids.at[pl.ds(off, _K)], langx, sem))

    def start_ids(sc, b, tokx, langx, sem):
        for d in id_descs(sc, b, tokx, langx, sem):
            d.start()

    def wait_ids(sc, b, tokx, langx, sem):
        for d in id_descs(sc, b, tokx, langx, sem):
            d.wait()

    def gather_desc(tokx, buf, sem):
        return pltpu.make_async_copy(w_tok.at[tokx], buf, sem)

    def out_desc(sc, b, buf, sem):
        t0 = b * S + wbase + sc * _K
        return pltpu.make_async_copy(buf, out.at[pl.ds(t0, _K)], sem)

    def pe_desc(sc, sem):
        return pltpu.make_async_copy(pe.at[pl.ds(wbase + sc * _K, _K)], pebuf, sem)

    pltpu.sync_copy(lang_tab, lang_v)
    pe_desc(0, sem_pe).start()
    copy_ids(0, 0, toka, langa)
    gather_desc(toka, buf_a, sem_ga).start()
    start_ids(0, 1, tokb, langb, sem_ib)

    def compute(buf_x, langx):
        def splat_grp(g, _):
            lvec = langx[pl.ds(g * _L, _L)]
            for i16 in range(_L):
                lidsplat.at[g * _L + i16][:] = _lane_splat(lvec, i16)
            return _

        lax.fori_loop(0, _K // _L, splat_grp, None)

        for jb in range(_NJ // _JB):
            rows = [
                [lang_v.at[l][pl.ds((jb * _JB + j) * _L, _L)] for l in range(NUM_LANG)]
                for j in range(_JB)
            ]

            def tok_loop(i, _, jb=jb, rows=rows):
                lid = lidsplat.at[i][:]
                m0 = lid == 0
                m1 = lid == 1
                m2 = lid == 2
                for j in range(_JB):
                    jj = jb * _JB + j
                    t = buf_x.at[i][pl.ds(jj * _L, _L)]
                    p = pebuf.at[i][pl.ds(jj * _L, _L)]
                    r = jnp.where(
                        m0, rows[j][0],
                        jnp.where(m1, rows[j][1],
                                  jnp.where(m2, rows[j][2], rows[j][3])),
                    )
                    buf_x.at[i][pl.ds(jj * _L, _L)] = t * SCALE + p + r
                return _

            lax.fori_loop(0, _K, tok_loop, None)

    def step(sc, b,
             buf_x, tokx, langx, sem_gx, sem_ox, sem_ix,
             buf_y, toky, langy, sem_gy, sem_oy, sem_iy):
        if b == 0:
            @pl.when(sc != 0)
            def _wait_oy():
                out_desc(sc - 1, B - 1, buf_y, sem_oy).wait()

            wait_ids(sc, 1, toky, langy, sem_iy)
            gather_desc(toky, buf_y, sem_gy).start()
        elif b < B - 1:
            out_desc(sc, b - 1, buf_y, sem_oy).wait()
            wait_ids(sc, b + 1, toky, langy, sem_iy)
            gather_desc(toky, buf_y, sem_gy).start()
        else:
            @pl.when(sc + 1 < _NSC)
            def _next_sc():
                out_desc(sc, b - 1, buf_y, sem_oy).wait()
                wait_ids(sc + 1, 0, toky, langy, sem_iy)
                gather_desc(toky, buf_y, sem_gy).start()

        gather_desc(tokx, buf_x, sem_gx).wait()

        if b == 0:
            pe_desc(sc, sem_pe).wait()

        compute(buf_x, langx)

        if b < 2:
            start_ids(sc, b + 2, tokx, langx, sem_ix)
        else:
            @pl.when(sc + 1 < _NSC)
            def _ids_next_sc():
                start_ids(sc + 1, b - 2, tokx, langx, sem_ix)

        if b == B - 1:
            @pl.when(sc + 1 < _NSC)
            def _pe_next():
                pe_desc(sc + 1, sem_pe).start()

        out_desc(sc, b, buf_x, sem_ox).start()

    def sc_loop(sc, _):
        step(sc, 0, buf_a, toka, langa, sem_ga, sem_oa, sem_ia,
             buf_b, tokb, langb, sem_gb, sem_ob, sem_ib)
        step(sc, 1, buf_b, tokb, langb, sem_gb, sem_ob, sem_ib,
             buf_a, toka, langa, sem_ga, sem_oa, sem_ia)
        step(sc, 2, buf_a, toka, langa, sem_ga, sem_oa, sem_ia,
             buf_b, tokb, langb, sem_gb, sem_ob, sem_ib)
        step(sc, 3, buf_b, tokb, langb, sem_gb, sem_ob, sem_ib,
             buf_a, toka, langa, sem_ga, sem_oa, sem_ia)
        return _

    lax.fori_loop(0, _NSC, sc_loop, None)

    out_desc(_NSC - 1, 2, buf_a, sem_oa).wait()
    out_desc(_NSC - 1, 3, buf_b, sem_ob).wait()


def kernel(token_ids, lang_ids, W_tok, W_lang, W_proj):
    lang_tab = _lang_tab(W_lang, W_proj)
    tok_flat = token_ids.reshape(-1).astype(jnp.int32)
    lang_flat = lang_ids.reshape(-1).astype(jnp.int32)
    pe = jnp.asarray(_PE[:S])
    out = _sc_embed(tok_flat, lang_flat, W_tok, lang_tab, pe)
    return out.reshape(B, S, D_MODEL)


# speedup vs baseline: 1.9529x; 1.0017x over previous
import functools
import math

import jax
import jax.numpy as jnp
import numpy as np
from jax import lax
from jax.experimental import pallas as pl
from jax.experimental.pallas import tpu as pltpu
from jax.experimental.pallas import tpu_sc as plsc

VOCAB = 100000
D_MODEL = 768
NUM_LANG = 4
LANG_DIM = 32
MAX_LEN = 4096
B = 4
S = 4096
SCALE = math.sqrt(D_MODEL)

_NW = 32
_SPW = S // _NW
_K = 32
_NSC = _SPW // _K
_L = 16
_NJ = D_MODEL // _L
_JB = 8


def _pe_np():
    pos = np.arange(MAX_LEN, dtype=np.float32)[:, None]
    div = np.exp(
        np.arange(0, D_MODEL, 2, dtype=np.float32)
        * np.float32(-math.log(10000.0) / D_MODEL)
    ).astype(np.float32)
    pe = np.zeros((MAX_LEN, D_MODEL), dtype=np.float32)
    pe[:, 0::2] = np.sin(pos * div)
    pe[:, 1::2] = np.cos(pos * div)
    return pe


_PE = _pe_np()
_GATHER_DN = lax.GatherDimensionNumbers(
    offset_dims=(), collapsed_slice_dims=(0,), start_index_map=(0,)
)


def _lane_splat(vec, lane):
    idx = jnp.full((16, 1), lane, jnp.int32)
    return lax.gather(
        vec, idx, _GATHER_DN, slice_sizes=(1,),
        mode=lax.GatherScatterMode.PROMISE_IN_BOUNDS,
    )


_mesh = plsc.VectorSubcoreMesh(core_axis_name="c", subcore_axis_name="s")


_NQ = B * _NSC


@functools.partial(
    pl.kernel,
    mesh=_mesh,
    out_type=jax.ShapeDtypeStruct((B * S, D_MODEL), jnp.float32),
    scratch_types=[
        pltpu.VMEM((_K,), jnp.int32),
        pltpu.VMEM((_K,), jnp.int32),
        pltpu.VMEM((_K,), jnp.int32),
        pltpu.VMEM((_K,), jnp.int32),
        pltpu.VMEM((_K, D_MODEL), jnp.float32),
        pltpu.VMEM((_K, D_MODEL), jnp.float32),
        pltpu.VMEM((_K, D_MODEL), jnp.float32),
        pltpu.VMEM((NUM_LANG, D_MODEL), jnp.float32),
        pltpu.VMEM((_K, _L), jnp.int32),
        pltpu.VMEM((NUM_LANG, LANG_DIM), jnp.float32),
        pltpu.SemaphoreType.DMA,
        pltpu.SemaphoreType.DMA,
        pltpu.SemaphoreType.DMA,
        pltpu.SemaphoreType.DMA,
        pltpu.SemaphoreType.DMA,
        pltpu.SemaphoreType.DMA,
        pltpu.SemaphoreType.DMA,
    ],
)
def _sc_embed(tok_ids, lang_ids, w_tok, w_lang, w_projt, pe, out,
              toka, tokb, langa, langb, buf_a, buf_b, pebuf, lang_v, lidsplat,
              wl_v,
              sem_ga, sem_gb, sem_oa, sem_ob, sem_pe, sem_ia, sem_ib):
    cid = lax.axis_index("c")
    sid = lax.axis_index("s")
    wid = sid * 2 + cid
    wbase = wid * _SPW

    def id_off(sc, b):
        return b * S + wbase + sc * _K

    def copy_ids(sc, b, tokx, langx):
        off = id_off(sc, b)
        pltpu.sync_copy(tok_ids.at[pl.ds(off, _K)], tokx)
        pltpu.sync_copy(lang_ids.at[pl.ds(off, _K)], langx)

    def id_descs(sc, b, tokx, langx, sem):
        off = id_off(sc, b)
        return (pltpu.make_async_copy(tok_ids.at[pl.ds(off, _K)], tokx, sem),
                pltpu.make_async_copy(lang_ids.at[pl.ds(off, _K)], langx, sem))

    def start_ids(sc, b, tokx, langx, sem):
        for d in id_descs(sc, b, tokx, langx, sem):
            d.start()

    def wait_ids(sc, b, tokx, langx, sem):
        for d in id_descs(sc, b, tokx, langx, sem):
            d.wait()

    def gather_desc(tokx, buf, sem):
        return pltpu.make_async_copy(w_tok.at[tokx], buf, sem)

    def out_desc(sc, b, buf, sem):
        t0 = b * S + wbase + sc * _K
        return pltpu.make_async_copy(buf, out.at[pl.ds(t0, _K)], sem)

    def pe_desc(sc, sem):
        return pltpu.make_async_copy(pe.at[pl.ds(wbase + sc * _K, _K)], pebuf, sem)

    pe_desc(0, sem_pe).start()
    copy_ids(0, 0, toka, langa)
    gather_desc(toka, buf_a, sem_ga).start()
    start_ids(0, 1, tokb, langb, sem_ib)

    pltpu.sync_copy(w_lang, wl_v)
    pltpu.sync_copy(w_projt, buf_b)
    wl_lo = [wl_v.at[l][pl.ds(0, _L)] for l in range(NUM_LANG)]
    wl_hi = [wl_v.at[l][pl.ds(_L, _L)] for l in range(NUM_LANG)]
    for jg in range(_NJ // _JB):
        def mm_k(k, acc, jg=jg):
            wp = [buf_b.at[k][pl.ds((jg * _JB + j) * _L, _L)] for j in range(_JB)]
            lane = lax.rem(k, _L)
            in_lo = k < _L
            sp = [
                _lane_splat(jnp.where(in_lo, wl_lo[l], wl_hi[l]), lane)
                for l in range(NUM_LANG)
            ]
            return tuple(
                acc[l * _JB + j] + sp[l] * wp[j]
                for l in range(NUM_LANG)
                for j in range(_JB)
            )

        zero = jnp.zeros((_L,), jnp.float32)
        accs = lax.fori_loop(0, LANG_DIM, mm_k, (zero,) * (NUM_LANG * _JB))
        for l in range(NUM_LANG):
            for j in range(_JB):
                lang_v.at[l][pl.ds((jg * _JB + j) * _L, _L)] = accs[l * _JB + j]

    def compute(buf_x, langx):
        def splat_grp(g, _):
            lvec = langx[pl.ds(g * _L, _L)]
            for i16 in range(_L):
                lidsplat.at[g * _L + i16][:] = _lane_splat(lvec, i16)
            return _

        lax.fori_loop(0, _K // _L, splat_grp, None)

        for jb in range(_NJ // _JB):
            rows = [
                [lang_v.at[l][pl.ds((jb * _JB + j) * _L, _L)] for l in range(NUM_LANG)]
                for j in range(_JB)
            ]

            def tok_loop(i, _, jb=jb, rows=rows):
                lid = lidsplat.at[i][:]
                m0 = lid == 0
                m1 = lid == 1
                m2 = lid == 2
                for j in range(_JB):
                    jj = jb * _JB + j
                    t = buf_x.at[i][pl.ds(jj * _L, _L)]
                    p = pebuf.at[i][pl.ds(jj * _L, _L)]
                    r = jnp.where(
                        m0, rows[j][0],
                        jnp.where(m1, rows[j][1],
                                  jnp.where(m2, rows[j][2], rows[j][3])),
                    )
                    buf_x.at[i][pl.ds(jj * _L, _L)] = t * SCALE + p + r
                return _

            lax.fori_loop(0, _K, tok_loop, None)

    def step(sc, b,
             buf_x, tokx, langx, sem_gx, sem_ox, sem_ix,
             buf_y, toky, langy, sem_gy, sem_oy, sem_iy):
        if b == 0:
            @pl.when(sc != 0)
            def _wait_oy():
                out_desc(sc - 1, B - 1, buf_y, sem_oy).wait()

            wait_ids(sc, 1, toky, langy, sem_iy)
            gather_desc(toky, buf_y, sem_gy).start()
        elif b < B - 1:
            out_desc(sc, b - 1, buf_y, sem_oy).wait()
            wait_ids(sc, b + 1, toky, langy, sem_iy)
            gather_desc(toky, buf_y, sem_gy).start()
        else:
            @pl.when(sc + 1 < _NSC)
            def _next_sc():
                out_desc(sc, b - 1, buf_y, sem_oy).wait()
                wait_ids(sc + 1, 0, toky, langy, sem_iy)
                gather_desc(toky, buf_y, sem_gy).start()

        gather_desc(tokx, buf_x, sem_gx).wait()

        if b == 0:
            pe_desc(sc, sem_pe).wait()

        compute(buf_x, langx)

        if b < 2:
            start_ids(sc, b + 2, tokx, langx, sem_ix)
        else:
            @pl.when(sc + 1 < _NSC)
            def _ids_next_sc():
                start_ids(sc + 1, b - 2, tokx, langx, sem_ix)

        if b == B - 1:
            @pl.when(sc + 1 < _NSC)
            def _pe_next():
                pe_desc(sc + 1, sem_pe).start()

        out_desc(sc, b, buf_x, sem_ox).start()

    def sc_loop(sc, _):
        step(sc, 0, buf_a, toka, langa, sem_ga, sem_oa, sem_ia,
             buf_b, tokb, langb, sem_gb, sem_ob, sem_ib)
        step(sc, 1, buf_b, tokb, langb, sem_gb, sem_ob, sem_ib,
             buf_a, toka, langa, sem_ga, sem_oa, sem_ia)
        step(sc, 2, buf_a, toka, langa, sem_ga, sem_oa, sem_ia,
             buf_b, tokb, langb, sem_gb, sem_ob, sem_ib)
        step(sc, 3, buf_b, tokb, langb, sem_gb, sem_ob, sem_ib,
             buf_a, toka, langa, sem_ga, sem_oa, sem_ia)
        return _

    lax.fori_loop(0, _NSC, sc_loop, None)

    out_desc(_NSC - 1, 2, buf_a, sem_oa).wait()
    out_desc(_NSC - 1, 3, buf_b, sem_ob).wait()


def kernel(token_ids, lang_ids, W_tok, W_lang, W_proj):
    tok_flat = token_ids.reshape(-1).astype(jnp.int32)
    lang_flat = lang_ids.reshape(-1).astype(jnp.int32)
    pe = jnp.asarray(_PE[:S])
    w_projt = W_proj.T
    out = _sc_embed(tok_flat, lang_flat, W_tok, W_lang, w_projt, pe)
    return out.reshape(B, S, D_MODEL)


# speedup vs baseline: 1.9532x; 1.0002x over previous
import functools
import math

import jax
import jax.numpy as jnp
import numpy as np
from jax import lax
from jax.experimental import pallas as pl
from jax.experimental.pallas import tpu as pltpu
from jax.experimental.pallas import tpu_sc as plsc

VOCAB = 100000
D_MODEL = 768
NUM_LANG = 4
LANG_DIM = 32
MAX_LEN = 4096
B = 4
S = 4096
SCALE = math.sqrt(D_MODEL)

_NW = 32
_SPW = S // _NW
_K = 32
_NSC = _SPW // _K
_L = 16
_NJ = D_MODEL // _L
_JB = 8


def _pe_np():
    pos = np.arange(MAX_LEN, dtype=np.float32)[:, None]
    div = np.exp(
        np.arange(0, D_MODEL, 2, dtype=np.float32)
        * np.float32(-math.log(10000.0) / D_MODEL)
    ).astype(np.float32)
    pe = np.zeros((MAX_LEN, D_MODEL), dtype=np.float32)
    pe[:, 0::2] = np.sin(pos * div)
    pe[:, 1::2] = np.cos(pos * div)
    return pe


_PE = _pe_np()
_GATHER_DN = lax.GatherDimensionNumbers(
    offset_dims=(), collapsed_slice_dims=(0,), start_index_map=(0,)
)


def _lane_splat(vec, lane):
    idx = jnp.full((16, 1), lane, jnp.int32)
    return lax.gather(
        vec, idx, _GATHER_DN, slice_sizes=(1,),
        mode=lax.GatherScatterMode.PROMISE_IN_BOUNDS,
    )


_mesh = plsc.VectorSubcoreMesh(core_axis_name="c", subcore_axis_name="s")


_NQ = B * _NSC


@functools.partial(
    pl.kernel,
    mesh=_mesh,
    out_type=jax.ShapeDtypeStruct((B * S, D_MODEL), jnp.float32),
    scratch_types=[
        pltpu.VMEM((_K,), jnp.int32),
        pltpu.VMEM((_K,), jnp.int32),
        pltpu.VMEM((_K,), jnp.int32),
        pltpu.VMEM((_K,), jnp.int32),
        pltpu.VMEM((_K,), jnp.int32),
        pltpu.VMEM((_K,), jnp.int32),
        pltpu.VMEM((_K, D_MODEL), jnp.float32),
        pltpu.VMEM((_K, D_MODEL), jnp.float32),
        pltpu.VMEM((_K, D_MODEL), jnp.float32),
        pltpu.VMEM((_K, D_MODEL), jnp.float32),
        pltpu.VMEM((NUM_LANG, D_MODEL), jnp.float32),
        pltpu.VMEM((_K, _L), jnp.int32),
        pltpu.VMEM((NUM_LANG, LANG_DIM), jnp.float32),
        pltpu.SemaphoreType.DMA,
        pltpu.SemaphoreType.DMA,
        pltpu.SemaphoreType.DMA,
        pltpu.SemaphoreType.DMA,
        pltpu.SemaphoreType.DMA,
        pltpu.SemaphoreType.DMA,
        pltpu.SemaphoreType.DMA,
        pltpu.SemaphoreType.DMA,
        pltpu.SemaphoreType.DMA,
        pltpu.SemaphoreType.DMA,
    ],
)
def _sc_embed(tok_ids, lang_ids, w_tok, w_lang, w_projt, pe, out,
              toka, tokb, tokc, langa, langb, langc,
              buf_a, buf_b, buf_c, pebuf, lang_v, lidsplat, wl_v,
              sem_ga, sem_gb, sem_gc, sem_oa, sem_ob, sem_oc,
              sem_pe, sem_ia, sem_ib, sem_ic):
    cid = lax.axis_index("c")
    sid = lax.axis_index("s")
    wid = sid * 2 + cid
    wbase = wid * _SPW

    def q_off(q):
        sc = q // B
        b = q - sc * B
        return b * S + wbase + sc * _K

    def id_descs(q, tokx, langx, sem):
        off = q_off(q)
        return (pltpu.make_async_copy(tok_ids.at[pl.ds(off, _K)], tokx, sem),
                pltpu.make_async_copy(lang_ids.at[pl.ds(off, _K)], langx, sem))

    def copy_ids(q, tokx, langx):
        off = q_off(q)
        pltpu.sync_copy(tok_ids.at[pl.ds(off, _K)], tokx)
        pltpu.sync_copy(lang_ids.at[pl.ds(off, _K)], langx)

    def start_ids(q, tokx, langx, sem):
        for d in id_descs(q, tokx, langx, sem):
            d.start()

    def wait_ids(q, tokx, langx, sem):
        for d in id_descs(q, tokx, langx, sem):
            d.wait()

    def gather_desc(tokx, buf, sem):
        return pltpu.make_async_copy(w_tok.at[tokx], buf, sem)

    def out_desc(q, buf, sem):
        return pltpu.make_async_copy(buf, out.at[pl.ds(q_off(q), _K)], sem)

    def pe_desc(sc, sem):
        return pltpu.make_async_copy(pe.at[pl.ds(wbase + sc * _K, _K)], pebuf, sem)

    def when(cond):
        def deco(fn):
            if isinstance(cond, bool):
                if cond:
                    fn()
            else:
                pl.when(cond)(fn)
        return deco

    pe_desc(0, sem_pe).start()
    copy_ids(0, toka, langa)
    gather_desc(toka, buf_a, sem_ga).start()
    copy_ids(1, tokb, langb)
    gather_desc(tokb, buf_b, sem_gb).start()
    start_ids(2, tokc, langc, sem_ic)

    pltpu.sync_copy(w_lang, wl_v)
    pltpu.sync_copy(w_projt, buf_c)
    wl_lo = [wl_v.at[l][pl.ds(0, _L)] for l in range(NUM_LANG)]
    wl_hi = [wl_v.at[l][pl.ds(_L, _L)] for l in range(NUM_LANG)]
    for jg in range(_NJ // _JB):
        def mm_k(k, acc, jg=jg):
            wp = [buf_c.at[k][pl.ds((jg * _JB + j) * _L, _L)] for j in range(_JB)]
            lane = lax.rem(k, _L)
            in_lo = k < _L
            sp = [
                _lane_splat(jnp.where(in_lo, wl_lo[l], wl_hi[l]), lane)
                for l in range(NUM_LANG)
            ]
            return tuple(
                acc[l * _JB + j] + sp[l] * wp[j]
                for l in range(NUM_LANG)
                for j in range(_JB)
            )

        zero = jnp.zeros((_L,), jnp.float32)
        accs = lax.fori_loop(0, LANG_DIM, mm_k, (zero,) * (NUM_LANG * _JB))
        for l in range(NUM_LANG):
            for j in range(_JB):
                lang_v.at[l][pl.ds((jg * _JB + j) * _L, _L)] = accs[l * _JB + j]

    def compute(buf_x, langx):
        def splat_grp(g, _):
            lvec = langx[pl.ds(g * _L, _L)]
            for i16 in range(_L):
                lidsplat.at[g * _L + i16][:] = _lane_splat(lvec, i16)
            return _

        lax.fori_loop(0, _K // _L, splat_grp, None)

        for jb in range(_NJ // _JB):
            rows = [
                [lang_v.at[l][pl.ds((jb * _JB + j) * _L, _L)] for l in range(NUM_LANG)]
                for j in range(_JB)
            ]

            def tok_loop(i, _, jb=jb, rows=rows):
                lid = lidsplat.at[i][:]
                m0 = lid == 0
                m1 = lid == 1
                m2 = lid == 2
                for j in range(_JB):
                    jj = jb * _JB + j
                    t = buf_x.at[i][pl.ds(jj * _L, _L)]
                    p = pebuf.at[i][pl.ds(jj * _L, _L)]
                    r = jnp.where(
                        m0, rows[j][0],
                        jnp.where(m1, rows[j][1],
                                  jnp.where(m2, rows[j][2], rows[j][3])),
                    )
                    buf_x.at[i][pl.ds(jj * _L, _L)] = t * SCALE + p + r
                return _

            lax.fori_loop(0, _K, tok_loop, None)

    bufs = (buf_a, buf_b, buf_c)
    toks = (toka, tokb, tokc)
    langs = (langa, langb, langc)
    gsems = (sem_ga, sem_gb, sem_gc)
    osems = (sem_oa, sem_ob, sem_oc)
    isems = (sem_ia, sem_ib, sem_ic)

    def step3(q, r):
        nxt = (r + 2) % 3

        @when((q >= 1) & (q <= _NQ - 3))
        def _recycle():
            out_desc(q - 1, bufs[nxt], osems[nxt]).wait()

        @when(q <= _NQ - 3)
        def _issue_gather2():
            wait_ids(q + 2, toks[nxt], langs[nxt], isems[nxt])
            gather_desc(toks[nxt], bufs[nxt], gsems[nxt]).start()

        gather_desc(toks[r], bufs[r], gsems[r]).wait()

        @when(q % B == 0)
        def _pe_wait():
            pe_desc(q // B, sem_pe).wait()

        compute(bufs[r], langs[r])

        @when(q <= _NQ - 4)
        def _ids3():
            start_ids(q + 3, toks[r], langs[r], isems[r])

        @when((q % B == B - 1) & (q // B + 1 < _NSC))
        def _pe_next():
            pe_desc(q // B + 1, sem_pe).start()

        out_desc(q, bufs[r], osems[r]).start()

    def tri_loop(t, _):
        q0 = 3 * t
        step3(q0, 0)
        step3(q0 + 1, 1)
        step3(q0 + 2, 2)
        return _

    lax.fori_loop(0, (_NQ - 1) // 3, tri_loop, None)
    step3(_NQ - 1, (_NQ - 1) % 3)

    out_desc(_NQ - 3, bufs[(_NQ - 3) % 3], osems[(_NQ - 3) % 3]).wait()
    out_desc(_NQ - 2, bufs[(_NQ - 2) % 3], osems[(_NQ - 2) % 3]).wait()
    out_desc(_NQ - 1, bufs[(_NQ - 1) % 3], osems[(_NQ - 1) % 3]).wait()


def kernel(token_ids, lang_ids, W_tok, W_lang, W_proj):
    tok_flat = token_ids.reshape(-1).astype(jnp.int32)
    lang_flat = lang_ids.reshape(-1).astype(jnp.int32)
    pe = jnp.asarray(_PE[:S])
    w_projt = W_proj.T
    out = _sc_embed(tok_flat, lang_flat, W_tok, W_lang, w_projt, pe)
    return out.reshape(B, S, D_MODEL)
